# Initial kernel scaffold; baseline (speedup 1.0000x reference)
import functools

import jax
import jax.numpy as jnp
from jax import lax
from jax.experimental import pallas as pl
from jax.experimental.pallas import tpu as pltpu, tpu_sc as plsc

N = 50000
D = 64
DH = 32
NPAD = 50176
PT = NPAD // 16
E = 800000
EPAD = 802816
CH = 128
ROWS_ALL = EPAD // CH
ROWS_W = ROWS_ALL // 32
ROWS_T = ROWS_ALL // 16
ET = EPAD // 16
BR = 512
GRID = NPAD // BR

_mesh = plsc.VectorSubcoreMesh(core_axis_name="c", subcore_axis_name="s")


@functools.partial(
    pl.kernel,
    out_type=jax.ShapeDtypeStruct((2, NPAD), jnp.float32),
    mesh=_mesh,
    scratch_types=[
        pltpu.VMEM((ROWS_W, CH), jnp.int32),
        pltpu.VMEM((CH,), jnp.float32),
        pltpu.VMEM_SHARED((NPAD,), jnp.float32),
    ],
)
def _deg_sc(dst_hbm, z1_hbm, out_hbm, dstv, ones, counts):
    c = lax.axis_index("c")
    s = lax.axis_index("s")
    w = c * 16 + s
    pltpu.sync_copy(z1_hbm, counts.at[pl.ds(s * PT, PT)])
    pltpu.sync_copy(dst_hbm.at[pl.ds(w * ROWS_W, ROWS_W)], dstv)
    for k in range(CH // 16):
        ones[pl.ds(k * 16, 16)] = jnp.ones((16,), jnp.float32)
    plsc.subcore_barrier()

    def step(i, carry):
        pltpu.sync_copy(ones, counts.at[dstv.at[i]], add=True)
        return carry

    lax.fori_loop(0, ROWS_W, step, 0)
    plsc.subcore_barrier()
    pltpu.sync_copy(counts.at[pl.ds(s * PT, PT)],
                    out_hbm.at[c, pl.ds(s * PT, PT)])


@functools.partial(
    pl.kernel,
    out_type=jax.ShapeDtypeStruct((2, NPAD, DH), jnp.float32),
    mesh=_mesh,
    scratch_types=[
        pltpu.VMEM((ET,), jnp.int32),
        pltpu.VMEM((ROWS_T, CH), jnp.int32),
        pltpu.VMEM((CH, DH), jnp.float32),
        pltpu.VMEM((CH, DH), jnp.float32),
        pltpu.VMEM_SHARED((NPAD, DH), jnp.float32),
        pltpu.SemaphoreType.DMA,
        pltpu.SemaphoreType.DMA,
    ],
)
def _edge_sc(src_hbm, dst_hbm, hpa_hbm, hpb_hbm, z2_hbm, out_hbm,
             srcv, dstv, buf0, buf1, acc, sem0, sem1):
    c = lax.axis_index("c")
    s = lax.axis_index("s")
    pltpu.sync_copy(z2_hbm, acc.at[pl.ds(s * PT, PT)])
    pltpu.sync_copy(src_hbm.at[pl.ds(s * ET, ET)], srcv)
    pltpu.sync_copy(dst_hbm.at[pl.ds(s * ROWS_T, ROWS_T)], dstv)
    plsc.subcore_barrier()

    def issue(i, buf, sem):
        sl = srcv.at[pl.ds(i * CH, CH)]

        @pl.when(c == 0)
        def _():
            pltpu.async_copy(hpa_hbm.at[sl], buf, sem)

        @pl.when(c == 1)
        def _():
            pltpu.async_copy(hpb_hbm.at[sl], buf, sem)

    def drain(buf, sem):
        pltpu.make_async_copy(hpa_hbm.at[pl.ds(0, CH)], buf, sem).wait()

    issue(0, buf0, sem0)

    def step(j, carry):
        i0 = 2 * j
        i1 = 2 * j + 1
        drain(buf0, sem0)
        issue(i1, buf1, sem1)
        pltpu.sync_copy(buf0, acc.at[dstv.at[i0]], add=True)
        drain(buf1, sem1)

        @pl.when(i1 + 1 < ROWS_T)
        def _():
            issue(i1 + 1, buf0, sem0)

        pltpu.sync_copy(buf1, acc.at[dstv.at[i1]], add=True)
        return carry

    lax.fori_loop(0, ROWS_T // 2, step, 0)
    plsc.subcore_barrier()
    pltpu.sync_copy(acc.at[pl.ds(s * PT, PT)],
                    out_hbm.at[c, pl.ds(s * PT, PT)])


def _mm_body(x_ref, w_ref, o_ref):
    o_ref[...] = jnp.dot(x_ref[...], w_ref[...],
                         preferred_element_type=jnp.float32)


def _mm(xp, W):
    return pl.pallas_call(
        _mm_body,
        grid=(GRID,),
        in_specs=[pl.BlockSpec((BR, D), lambda i: (i, 0)),
                  pl.BlockSpec((D, D), lambda i: (0, 0))],
        out_specs=pl.BlockSpec((BR, D), lambda i: (i, 0)),
        out_shape=jax.ShapeDtypeStruct((NPAD, D), jnp.float32),
    )(xp, W)


def _scale_body(c0_ref, c1_ref, h_ref, dinv_ref, hp_ref):
    deg = c0_ref[...] + c1_ref[...] + 1.0
    dinv = lax.rsqrt(deg)
    dinv_ref[...] = dinv
    hp = h_ref[...] * dinv[:, None]
    hp_ref[0, :, :] = hp[:, :DH]
    hp_ref[1, :, :] = hp[:, DH:]


def _scale(c0, c1, h):
    return pl.pallas_call(
        _scale_body,
        grid=(GRID,),
        in_specs=[pl.BlockSpec((BR,), lambda i: (i,)),
                  pl.BlockSpec((BR,), lambda i: (i,)),
                  pl.BlockSpec((BR, D), lambda i: (i, 0))],
        out_specs=[pl.BlockSpec((BR,), lambda i: (i,)),
                   pl.BlockSpec((2, BR, DH), lambda i: (0, i, 0))],
        out_shape=[jax.ShapeDtypeStruct((NPAD,), jnp.float32),
                   jax.ShapeDtypeStruct((2, NPAD, DH), jnp.float32)],
    )(c0, c1, h)


def _epiA_body(s_ref, h_ref, dinv_ref, b_ref, o_ref, a_ref, q_ref):
    i = pl.program_id(0)
    sc = jnp.concatenate([s_ref[0], s_ref[1]], axis=-1)
    dinv = dinv_ref[...]
    out = (sc + h_ref[...] * dinv[:, None]) * dinv[:, None] + b_ref[...][None, :]
    o_ref[...] = out
    ridx = i * BR + lax.broadcasted_iota(jnp.int32, (BR, 1), 0)
    om = jnp.where(ridx < N, out, 0.0)
    ps = jnp.sum(om.reshape(8, BR // 8, D), axis=1)
    psq = jnp.sum((om * om).reshape(8, BR // 8, D), axis=1)

    @pl.when(i == 0)
    def _():
        a_ref[...] = ps
        q_ref[...] = psq

    @pl.when(i > 0)
    def _():
        a_ref[...] += ps
        q_ref[...] += psq


def _epiA(S, h, dinv, b):
    return pl.pallas_call(
        _epiA_body,
        grid=(GRID,),
        in_specs=[pl.BlockSpec((2, BR, DH), lambda i: (0, i, 0)),
                  pl.BlockSpec((BR, D), lambda i: (i, 0)),
                  pl.BlockSpec((BR,), lambda i: (i,)),
                  pl.BlockSpec((D,), lambda i: (0,))],
        out_specs=[pl.BlockSpec((BR, D), lambda i: (i, 0)),
                   pl.BlockSpec((8, D), lambda i: (0, 0)),
                   pl.BlockSpec((8, D), lambda i: (0, 0))],
        out_shape=[jax.ShapeDtypeStruct((NPAD, D), jnp.float32),
                   jax.ShapeDtypeStruct((8, D), jnp.float32),
                   jax.ShapeDtypeStruct((8, D), jnp.float32)],
    )(S, h, dinv, b)


def _epiB_body(o_ref, a_ref, q_ref, g_ref, be_ref, y_ref):
    tot = jnp.sum(a_ref[...], axis=0)
    totq = jnp.sum(q_ref[...], axis=0)
    mean = tot * (1.0 / N)
    var = totq * (1.0 / N) - mean * mean
    inv = lax.rsqrt(var + 1e-5) * g_ref[...]
    yv = (o_ref[...] - mean[None, :]) * inv[None, :] + be_ref[...][None, :]
    y_ref[...] = jnp.where(yv >= 0, yv, 0.01 * yv)


def _epiB(o, a, q, g, be):
    return pl.pallas_call(
        _epiB_body,
        grid=(GRID,),
        in_specs=[pl.BlockSpec((BR, D), lambda i: (i, 0)),
                  pl.BlockSpec((8, D), lambda i: (0, 0)),
                  pl.BlockSpec((8, D), lambda i: (0, 0)),
                  pl.BlockSpec((D,), lambda i: (0,)),
                  pl.BlockSpec((D,), lambda i: (0,))],
        out_specs=pl.BlockSpec((BR, D), lambda i: (i, 0)),
        out_shape=jax.ShapeDtypeStruct((NPAD, D), jnp.float32),
    )(o, a, q, g, be)


def kernel(x, edge_index, W1, b1, g1, be1, W2, b2, g2, be2):
    src = edge_index[0]
    dst = edge_index[1]
    pad = N + (jnp.arange(EPAD - E, dtype=jnp.int32) % (NPAD - N))
    srcp = jnp.concatenate([src, pad])
    dstp = jnp.concatenate([dst, pad]).reshape(ROWS_ALL, CH)
    xp = jnp.zeros((NPAD, D), jnp.float32).at[:N].set(x)
    z1 = jnp.zeros((PT,), jnp.float32)
    z2 = jnp.zeros((PT, DH), jnp.float32)

    counts = _deg_sc(dstp, z1)
    h1 = _mm(xp, W1)
    dinv, hp1 = _scale(counts[0], counts[1], h1)
    S1 = _edge_sc(srcp, dstp, hp1[0], hp1[1], z2)
    o1, a1, q1 = _epiA(S1, h1, dinv, b1)
    y1 = _epiB(o1, a1, q1, g1, be1)
    h2 = _mm(y1, W2)
    _, hp2 = _scale(counts[0], counts[1], h2)
    S2 = _edge_sc(srcp, dstp, hp2[0], hp2[1], z2)
    o2, a2, q2 = _epiA(S2, h2, dinv, b2)
    y2 = _epiB(o2, a2, q2, g2, be2)
    return y2[:N]


# speedup vs baseline: 4.4355x; 4.4355x over previous
import functools

import jax
import jax.numpy as jnp
from jax import lax
from jax.experimental import pallas as pl
from jax.experimental.pallas import tpu as pltpu, tpu_sc as plsc

N = 50000
D = 64
DH = 32
NPAD = 50176
PT = NPAD // 16
E = 800000
EPAD = 819200
CH = 128
ROWS_ALL = EPAD // CH
ROWS_W = ROWS_ALL // 32
ROWS_T = ROWS_ALL // 16
ET = EPAD // 16
BR = 512
GRID = NPAD // BR
NPASS = 4
R = NPAD // 4
AT = R // 16
ST = AT // 8
NQ = 2
DUMP = 64
ACCN = R + DUMP

_mesh = plsc.VectorSubcoreMesh(core_axis_name="c", subcore_axis_name="s")


EPW = EPAD // 32


@functools.partial(
    pl.kernel,
    out_type=jax.ShapeDtypeStruct((32 * NPAD,), jnp.float32),
    mesh=_mesh,
    compiler_params=pltpu.CompilerParams(use_tc_tiling_on_sc=False,
                                         needs_layout_passes=False),
    scratch_types=[
        pltpu.VMEM((EPW,), jnp.int32),
        pltpu.VMEM((NPAD,), jnp.float32),
    ],
)
def _deg_sc(dst_hbm, out_hbm, dstv, counts):
    c = lax.axis_index("c")
    s = lax.axis_index("s")
    w = c * 16 + s
    pltpu.sync_copy(dst_hbm.at[pl.ds(w * EPW, EPW)], dstv)

    def zstep(i, carry):
        counts[pl.ds(i * 16, 16)] = jnp.zeros((16,), jnp.float32)
        return carry

    lax.fori_loop(0, NPAD // 16, zstep, 0)
    ones = jnp.ones((16,), jnp.float32)

    def step(i, carry):
        idx = dstv[pl.ds(i * 16, 16)]
        plsc.addupdate_scatter(counts, [idx], ones)
        return carry

    lax.fori_loop(0, EPW // 16, step, 0)
    pltpu.sync_copy(counts, out_hbm.at[pl.ds(w * NPAD, NPAD)])


@functools.partial(
    pl.kernel,
    out_type=jax.ShapeDtypeStruct((4 * NPAD, 128), jnp.float32),
    mesh=_mesh,
    compiler_params=pltpu.CompilerParams(use_tc_tiling_on_sc=False,
                                         needs_layout_passes=False),
    scratch_types=[
        pltpu.VMEM((ET,), jnp.int32),
        pltpu.VMEM((2, CH), jnp.int32),
        pltpu.VMEM((2, CH), jnp.int32),
        pltpu.VMEM((2, CH), jnp.int32),
        pltpu.VMEM((2, CH, 128), jnp.float32),
        pltpu.VMEM((2, CH, DH), jnp.bfloat16),
        pltpu.VMEM((98, 128), jnp.float32),
        pltpu.VMEM((98, DH), jnp.bfloat16),
        pltpu.VMEM_SHARED((NQ * ACCN, DH), jnp.bfloat16),
        pltpu.SemaphoreType.DMA,
        pltpu.SemaphoreType.DMA,
    ],
)
def _edge_sc(src_hbm, dst_hbm, hp_hbm, out_hbm,
             srcv, dbuf, gbuf, ibuf, bufs, bbuf, stage, qtmp, acc,
             sem0, sem1):
    c = lax.axis_index("c")
    s = lax.axis_index("s")
    coff = c * NPAD
    pltpu.sync_copy(src_hbm.at[pl.ds(s * ET, ET)], srcv)

    def zwide(i, carry):
        for m in range(8):
            stage[i, pl.ds(m * 16, 16)] = jnp.zeros((16,), jnp.float32)
        return carry

    lax.fori_loop(0, 98, zwide, 0)

    def zq(i, carry):
        qtmp[i, :] = jnp.zeros((DH,), jnp.bfloat16)
        return carry

    def passbody(ph, carry):
        p = ph // 2
        h = ph - 2 * p
        lo = p * R
        cbase = h * (ROWS_T // 2)

        lax.fori_loop(0, ST, zq, 0)

        def zcopy(k, carry1):
            q = k // 8
            kk = k - q * 8
            pltpu.sync_copy(
                qtmp, acc.at[pl.ds(q * ACCN + s * AT + kk * ST, ST)])
            return carry1

        lax.fori_loop(0, 8 * NQ, zcopy, 0)
        plsc.subcore_barrier()

        def step(j, carry1):
            i0 = cbase + 2 * j
            ddesc = pltpu.async_copy(
                dst_hbm.at[pl.ds(s * ROWS_T + i0, 2)], dbuf, sem1)
            descs = []
            for k in range(2):
                for m in range(CH // 16):
                    v = srcv[pl.ds((i0 + k) * CH + m * 16, 16)]
                    gbuf[k, pl.ds(m * 16, 16)] = v + coff
                descs.append(pltpu.async_copy(
                    hp_hbm.at[gbuf.at[k]], bufs.at[k], sem0))
            ddesc.wait()
            for k in range(2):
                q = lax.rem(i0 + k, NQ)
                for m in range(CH // 16):
                    v = dbuf[k, pl.ds(m * 16, 16)]
                    vl = v - lo
                    inr = (vl >= 0) & (vl < R)
                    ibuf[k, pl.ds(m * 16, 16)] = q * ACCN + jnp.where(
                        inr, vl, R + (v & (DUMP - 1)))

            for k in range(2):
                descs[k].wait()
                def cvt(e, carry2, k=k):
                    a = bufs[k, e, pl.ds(0, 16)]
                    b = bufs[k, e, pl.ds(16, 16)]
                    bbuf[k, e, :] = plsc.pack(
                        a, b, format=plsc.PackFormat.INTERLEAVED)
                    return carry2

                lax.fori_loop(0, CH, cvt, 0)

            def scat(k, carry2):
                pltpu.sync_copy(bbuf.at[k], acc.at[ibuf.at[k]], add=True)
                return carry2

            lax.fori_loop(0, 2, scat, 0)
            return carry1

        lax.fori_loop(0, ROWS_T // 4, step, 0)
        plsc.subcore_barrier()

        def rcopy(k, carry1):
            for q in range(NQ):
                pltpu.sync_copy(
                    acc.at[pl.ds(q * ACCN + s * AT + k * ST, ST)], qtmp)

                def comb(i, carry2, q=q):
                    a, b = plsc.unpack(
                        qtmp[i, :], format=plsc.PackFormat.INTERLEAVED)
                    if q == 0:
                        stage[i, pl.ds(0, 16)] = a
                        stage[i, pl.ds(16, 16)] = b
                    else:
                        stage[i, pl.ds(0, 16)] = stage[i, pl.ds(0, 16)] + a
                        stage[i, pl.ds(16, 16)] = stage[i, pl.ds(16, 16)] + b
                    return carry2

                lax.fori_loop(0, ST, comb, 0)
            pltpu.sync_copy(
                stage,
                out_hbm.at[pl.ds((h * 2 + c) * NPAD + lo
                                 + s * AT + k * ST, ST)])
            return carry1

        lax.fori_loop(0, 8, rcopy, 0)
        return carry

    lax.fori_loop(0, 2 * NPASS, passbody, 0)


def _mm_body(x_ref, w_ref, o_ref):
    o_ref[...] = jnp.dot(x_ref[...], w_ref[...],
                         preferred_element_type=jnp.float32)


def _mm0_body(x_ref, w_ref, o_ref):
    i = pl.program_id(0)
    ridx = i * BR + lax.broadcasted_iota(jnp.int32, (BR, 1), 0)
    xv = jnp.where(ridx < N, x_ref[...], 0.0)
    o_ref[...] = jnp.dot(xv, w_ref[...], preferred_element_type=jnp.float32)


def _mm(xp, W, body=_mm_body):
    return pl.pallas_call(
        body,
        grid=(GRID,),
        in_specs=[pl.BlockSpec((BR, D), lambda i: (i, 0)),
                  pl.BlockSpec((D, D), lambda i: (0, 0))],
        out_specs=pl.BlockSpec((BR, D), lambda i: (i, 0)),
        out_shape=jax.ShapeDtypeStruct((NPAD, D), jnp.float32),
    )(xp, W)


def _scale_body(c_ref, h_ref, dinv_ref, hp_ref):
    deg = jnp.sum(c_ref[...], axis=0) + 1.0
    dinv = lax.rsqrt(deg)
    dinv_ref[...] = dinv
    hp = h_ref[...] * dinv[:, None]
    z = jnp.zeros((BR, 128 - DH), jnp.float32)
    hp_ref[0, :, :] = jnp.concatenate([hp[:, :DH], z], axis=-1)
    hp_ref[1, :, :] = jnp.concatenate([hp[:, DH:], z], axis=-1)


def _scale(c32, h):
    return pl.pallas_call(
        _scale_body,
        grid=(GRID,),
        in_specs=[pl.BlockSpec((32, BR), lambda i: (0, i)),
                  pl.BlockSpec((BR, D), lambda i: (i, 0))],
        out_specs=[pl.BlockSpec((BR,), lambda i: (i,)),
                   pl.BlockSpec((2, BR, 128), lambda i: (0, i, 0))],
        out_shape=[jax.ShapeDtypeStruct((NPAD,), jnp.float32),
                   jax.ShapeDtypeStruct((2, NPAD, 128), jnp.float32)],
    )(c32, h)


def _epiA_body(s_ref, h_ref, dinv_ref, b_ref, o_ref, a_ref, q_ref):
    i = pl.program_id(0)
    sc = jnp.concatenate(
        [s_ref[0, :, :DH] + s_ref[2, :, :DH],
         s_ref[1, :, :DH] + s_ref[3, :, :DH]], axis=-1)
    dinv = dinv_ref[...]
    out = (sc + h_ref[...] * dinv[:, None]) * dinv[:, None] + b_ref[...][None, :]
    o_ref[...] = out
    ridx = i * BR + lax.broadcasted_iota(jnp.int32, (BR, 1), 0)
    om = jnp.where(ridx < N, out, 0.0)
    ps = jnp.sum(om.reshape(8, BR // 8, D), axis=1)
    psq = jnp.sum((om * om).reshape(8, BR // 8, D), axis=1)

    @pl.when(i == 0)
    def _():
        a_ref[...] = ps
        q_ref[...] = psq

    @pl.when(i > 0)
    def _():
        a_ref[...] += ps
        q_ref[...] += psq


def _epiA(S, h, dinv, b):
    return pl.pallas_call(
        _epiA_body,
        grid=(GRID,),
        in_specs=[pl.BlockSpec((4, BR, 128), lambda i: (0, i, 0)),
                  pl.BlockSpec((BR, D), lambda i: (i, 0)),
                  pl.BlockSpec((BR,), lambda i: (i,)),
                  pl.BlockSpec((D,), lambda i: (0,))],
        out_specs=[pl.BlockSpec((BR, D), lambda i: (i, 0)),
                   pl.BlockSpec((8, D), lambda i: (0, 0)),
                   pl.BlockSpec((8, D), lambda i: (0, 0))],
        out_shape=[jax.ShapeDtypeStruct((NPAD, D), jnp.float32),
                   jax.ShapeDtypeStruct((8, D), jnp.float32),
                   jax.ShapeDtypeStruct((8, D), jnp.float32)],
    )(S, h, dinv, b)


def _epiB_body(o_ref, a_ref, q_ref, g_ref, be_ref, y_ref):
    tot = jnp.sum(a_ref[...], axis=0)
    totq = jnp.sum(q_ref[...], axis=0)
    mean = tot * (1.0 / N)
    var = totq * (1.0 / N) - mean * mean
    inv = lax.rsqrt(var + 1e-5) * g_ref[...]
    yv = (o_ref[...] - mean[None, :]) * inv[None, :] + be_ref[...][None, :]
    y_ref[...] = jnp.where(yv >= 0, yv, 0.01 * yv)


def _epiB(o, a, q, g, be, out_rows=NPAD):
    return pl.pallas_call(
        _epiB_body,
        grid=(GRID,),
        in_specs=[pl.BlockSpec((BR, D), lambda i: (i, 0)),
                  pl.BlockSpec((8, D), lambda i: (0, 0)),
                  pl.BlockSpec((8, D), lambda i: (0, 0)),
                  pl.BlockSpec((D,), lambda i: (0,)),
                  pl.BlockSpec((D,), lambda i: (0,))],
        out_specs=pl.BlockSpec((BR, D), lambda i: (i, 0)),
        out_shape=jax.ShapeDtypeStruct((out_rows, D), jnp.float32),
    )(o, a, q, g, be)


def kernel(x, edge_index, W1, b1, g1, be1, W2, b2, g2, be2):
    src = edge_index[0]
    dst = edge_index[1]
    pad = N + (jnp.arange(EPAD - E, dtype=jnp.int32) % (NPAD - N))
    srcp = jnp.concatenate([src, pad])
    dflat = jnp.concatenate([dst, pad])
    dstp = dflat.reshape(ROWS_ALL, CH)

    c32 = _deg_sc(dflat).reshape(32, NPAD)
    h1 = _mm(x, W1, body=_mm0_body)
    dinv, hp1 = _scale(c32, h1)
    S1 = _edge_sc(srcp, dstp, hp1.reshape(2 * NPAD, 128)).reshape(4, NPAD, 128)
    o1, a1, q1 = _epiA(S1, h1, dinv, b1)
    y1 = _epiB(o1, a1, q1, g1, be1)
    h2 = _mm(y1, W2)
    _, hp2 = _scale(c32, h2)
    S2 = _edge_sc(srcp, dstp, hp2.reshape(2 * NPAD, 128)).reshape(4, NPAD, 128)
    o2, a2, q2 = _epiA(S2, h2, dinv, b2)
    return _epiB(o2, a2, q2, g2, be2, out_rows=N)


# speedup vs baseline: 7.3343x; 1.6536x over previous
import functools

import jax
import jax.numpy as jnp
from jax import lax
from jax.experimental import pallas as pl
from jax.experimental.pallas import tpu as pltpu, tpu_sc as plsc

N = 50000
D = 64
DH = 32
NPAD = 50176
PT = NPAD // 16
E = 800000
EPAD = 819200
CH = 128
ROWS_ALL = EPAD // CH
ROWS_W = ROWS_ALL // 32
ROWS_T = ROWS_ALL // 16
ET = EPAD // 16
BR = 512
GRID = NPAD // BR
NQTR = 4
NPASS = 2
R = NPAD // 2
DUMP = 64
ACCN = R + DUMP
AT = R // 16
ST = 98

_mesh = plsc.VectorSubcoreMesh(core_axis_name="c", subcore_axis_name="s")


EPW = EPAD // 32


@functools.partial(
    pl.kernel,
    out_type=jax.ShapeDtypeStruct((32 * NPAD,), jnp.float32),
    mesh=_mesh,
    compiler_params=pltpu.CompilerParams(use_tc_tiling_on_sc=False,
                                         needs_layout_passes=False),
    scratch_types=[
        pltpu.VMEM((EPW,), jnp.int32),
        pltpu.VMEM((NPAD,), jnp.float32),
    ],
)
def _deg_sc(dst_hbm, out_hbm, dstv, counts):
    c = lax.axis_index("c")
    s = lax.axis_index("s")
    w = c * 16 + s
    pltpu.sync_copy(dst_hbm.at[pl.ds(w * EPW, EPW)], dstv)

    def zstep(i, carry):
        counts[pl.ds(i * 16, 16)] = jnp.zeros((16,), jnp.float32)
        return carry

    lax.fori_loop(0, NPAD // 16, zstep, 0)
    ones = jnp.ones((16,), jnp.float32)

    def step(i, carry):
        idx = dstv[pl.ds(i * 16, 16)]
        plsc.addupdate_scatter(counts, [idx], ones)
        return carry

    lax.fori_loop(0, EPW // 16, step, 0)
    pltpu.sync_copy(counts, out_hbm.at[pl.ds(w * NPAD, NPAD)])


@functools.partial(
    pl.kernel,
    out_type=jax.ShapeDtypeStruct((2 * NQTR * NPAD, 128), jnp.float32),
    mesh=_mesh,
    compiler_params=pltpu.CompilerParams(use_tc_tiling_on_sc=False,
                                         needs_layout_passes=False),
    scratch_types=[
        pltpu.VMEM((ET,), jnp.int32),
        pltpu.VMEM((2, CH), jnp.int32),
        pltpu.VMEM((2, CH), jnp.int32),
        pltpu.VMEM((2, CH), jnp.int32),
        pltpu.VMEM((2, CH, 128), jnp.float32),
        pltpu.VMEM((2, CH, DH), jnp.bfloat16),
        pltpu.VMEM((ST, 128), jnp.float32),
        pltpu.VMEM((ST, DH), jnp.bfloat16),
        pltpu.VMEM_SHARED((ACCN, DH), jnp.bfloat16),
        pltpu.SemaphoreType.DMA,
        pltpu.SemaphoreType.DMA,
    ],
)
def _edge_sc(src_hbm, dst_hbm, hp_hbm, out_hbm,
             srcv, dbuf, gbuf, ibuf, bufs, bbuf, stage, qtmp, acc,
             sem0, sem1):
    c = lax.axis_index("c")
    s = lax.axis_index("s")
    coff = c * NPAD
    pltpu.sync_copy(src_hbm.at[pl.ds(s * ET, ET)], srcv)

    def zwide(i, carry):
        for m in range(8):
            stage[i, pl.ds(m * 16, 16)] = jnp.zeros((16,), jnp.float32)
        return carry

    lax.fori_loop(0, ST, zwide, 0)

    def zq(i, carry):
        qtmp[i, :] = jnp.zeros((DH,), jnp.bfloat16)
        return carry

    lax.fori_loop(0, ST, zq, 0)

    def quarter(pq, carry):
        p = pq // NQTR
        qt = pq - p * NQTR
        lo = p * R
        cbase = qt * (ROWS_T // NQTR)

        def zcopy(k, carry1):
            pltpu.sync_copy(qtmp, acc.at[pl.ds(s * AT + k * ST, ST)])
            return carry1

        lax.fori_loop(0, AT // ST, zcopy, 0)
        plsc.subcore_barrier()

        def step(j, carry1):
            i0 = cbase + 2 * j
            ddesc = pltpu.async_copy(
                dst_hbm.at[pl.ds(s * ROWS_T + i0, 2)], dbuf, sem1)
            descs = []
            for k in range(2):
                for m in range(CH // 16):
                    v = srcv[pl.ds((i0 + k) * CH + m * 16, 16)]
                    gbuf[k, pl.ds(m * 16, 16)] = v + coff
                descs.append(pltpu.async_copy(
                    hp_hbm.at[gbuf.at[k]], bufs.at[k], sem0))
            ddesc.wait()
            for k in range(2):
                for m in range(CH // 16):
                    v = dbuf[k, pl.ds(m * 16, 16)]
                    vl = v - lo
                    inr = (vl >= 0) & (vl < R)
                    ibuf[k, pl.ds(m * 16, 16)] = jnp.where(
                        inr, vl, R + (v & (DUMP - 1)))
            for k in range(2):
                descs[k].wait()
                def cvt(e, carry2, k=k):
                    a = bufs[k, e, pl.ds(0, 16)]
                    b = bufs[k, e, pl.ds(16, 16)]
                    bbuf[k, e, :] = plsc.pack(
                        a, b, format=plsc.PackFormat.INTERLEAVED)
                    return carry2

                lax.fori_loop(0, CH, cvt, 0)

            def scat(k, carry2):
                pltpu.sync_copy(bbuf.at[k], acc.at[ibuf.at[k]], add=True)
                return carry2

            lax.fori_loop(0, 2, scat, 0)
            return carry1

        lax.fori_loop(0, ROWS_T // (2 * NQTR), step, 0)
        plsc.subcore_barrier()

        def rcopy(k, carry1):
            pltpu.sync_copy(acc.at[pl.ds(s * AT + k * ST, ST)], qtmp)

            def comb(i, carry2):
                a, b = plsc.unpack(
                    qtmp[i, :], format=plsc.PackFormat.INTERLEAVED)
                stage[i, pl.ds(0, 16)] = a
                stage[i, pl.ds(16, 16)] = b
                return carry2

            lax.fori_loop(0, ST, comb, 0)
            pltpu.sync_copy(
                stage,
                out_hbm.at[pl.ds((qt * 2 + c) * NPAD + lo
                                 + s * AT + k * ST, ST)])
            return carry1

        lax.fori_loop(0, AT // ST, rcopy, 0)
        lax.fori_loop(0, ST, zq, 0)
        return carry

    lax.fori_loop(0, NPASS * NQTR, quarter, 0)


def _mm_body(x_ref, w_ref, o_ref):
    o_ref[...] = jnp.dot(x_ref[...], w_ref[...],
                         preferred_element_type=jnp.float32)


def _mm0_body(x_ref, w_ref, o_ref):
    i = pl.program_id(0)
    ridx = i * BR + lax.broadcasted_iota(jnp.int32, (BR, 1), 0)
    xv = jnp.where(ridx < N, x_ref[...], 0.0)
    o_ref[...] = jnp.dot(xv, w_ref[...], preferred_element_type=jnp.float32)


def _mm(xp, W, body=_mm_body):
    return pl.pallas_call(
        body,
        grid=(GRID,),
        in_specs=[pl.BlockSpec((BR, D), lambda i: (i, 0)),
                  pl.BlockSpec((D, D), lambda i: (0, 0))],
        out_specs=pl.BlockSpec((BR, D), lambda i: (i, 0)),
        out_shape=jax.ShapeDtypeStruct((NPAD, D), jnp.float32),
    )(xp, W)


def _scale_body(c_ref, h_ref, dinv_ref, hp_ref):
    deg = jnp.sum(c_ref[...], axis=0) + 1.0
    dinv = lax.rsqrt(deg)
    dinv_ref[...] = dinv
    hp = h_ref[...] * dinv[:, None]
    z = jnp.zeros((BR, 128 - DH), jnp.float32)
    hp_ref[0, :, :] = jnp.concatenate([hp[:, :DH], z], axis=-1)
    hp_ref[1, :, :] = jnp.concatenate([hp[:, DH:], z], axis=-1)


def _scale(c32, h):
    return pl.pallas_call(
        _scale_body,
        grid=(GRID,),
        in_specs=[pl.BlockSpec((32, BR), lambda i: (0, i)),
                  pl.BlockSpec((BR, D), lambda i: (i, 0))],
        out_specs=[pl.BlockSpec((BR,), lambda i: (i,)),
                   pl.BlockSpec((2, BR, 128), lambda i: (0, i, 0))],
        out_shape=[jax.ShapeDtypeStruct((NPAD,), jnp.float32),
                   jax.ShapeDtypeStruct((2, NPAD, 128), jnp.float32)],
    )(c32, h)


def _epiA_body(s_ref, h_ref, dinv_ref, b_ref, o_ref, a_ref, q_ref):
    i = pl.program_id(0)
    s0 = ((s_ref[0, :, :DH] + s_ref[2, :, :DH])
          + (s_ref[4, :, :DH] + s_ref[6, :, :DH]))
    s1 = ((s_ref[1, :, :DH] + s_ref[3, :, :DH])
          + (s_ref[5, :, :DH] + s_ref[7, :, :DH]))
    sc = jnp.concatenate([s0, s1], axis=-1)
    dinv = dinv_ref[...]
    out = (sc + h_ref[...] * dinv[:, None]) * dinv[:, None] + b_ref[...][None, :]
    o_ref[...] = out
    ridx = i * BR + lax.broadcasted_iota(jnp.int32, (BR, 1), 0)
    om = jnp.where(ridx < N, out, 0.0)
    ps = jnp.sum(om.reshape(8, BR // 8, D), axis=1)
    psq = jnp.sum((om * om).reshape(8, BR // 8, D), axis=1)

    @pl.when(i == 0)
    def _():
        a_ref[...] = ps
        q_ref[...] = psq

    @pl.when(i > 0)
    def _():
        a_ref[...] += ps
        q_ref[...] += psq


def _epiA(S, h, dinv, b):
    return pl.pallas_call(
        _epiA_body,
        grid=(GRID,),
        in_specs=[pl.BlockSpec((8, BR, 128), lambda i: (0, i, 0)),
                  pl.BlockSpec((BR, D), lambda i: (i, 0)),
                  pl.BlockSpec((BR,), lambda i: (i,)),
                  pl.BlockSpec((D,), lambda i: (0,))],
        out_specs=[pl.BlockSpec((BR, D), lambda i: (i, 0)),
                   pl.BlockSpec((8, D), lambda i: (0, 0)),
                   pl.BlockSpec((8, D), lambda i: (0, 0))],
        out_shape=[jax.ShapeDtypeStruct((NPAD, D), jnp.float32),
                   jax.ShapeDtypeStruct((8, D), jnp.float32),
                   jax.ShapeDtypeStruct((8, D), jnp.float32)],
    )(S, h, dinv, b)


def _epiB_body(o_ref, a_ref, q_ref, g_ref, be_ref, y_ref):
    tot = jnp.sum(a_ref[...], axis=0)
    totq = jnp.sum(q_ref[...], axis=0)
    mean = tot * (1.0 / N)
    var = totq * (1.0 / N) - mean * mean
    inv = lax.rsqrt(var + 1e-5) * g_ref[...]
    yv = (o_ref[...] - mean[None, :]) * inv[None, :] + be_ref[...][None, :]
    y_ref[...] = jnp.where(yv >= 0, yv, 0.01 * yv)


def _epiB(o, a, q, g, be, out_rows=NPAD):
    return pl.pallas_call(
        _epiB_body,
        grid=(GRID,),
        in_specs=[pl.BlockSpec((BR, D), lambda i: (i, 0)),
                  pl.BlockSpec((8, D), lambda i: (0, 0)),
                  pl.BlockSpec((8, D), lambda i: (0, 0)),
                  pl.BlockSpec((D,), lambda i: (0,)),
                  pl.BlockSpec((D,), lambda i: (0,))],
        out_specs=pl.BlockSpec((BR, D), lambda i: (i, 0)),
        out_shape=jax.ShapeDtypeStruct((out_rows, D), jnp.float32),
    )(o, a, q, g, be)


def kernel(x, edge_index, W1, b1, g1, be1, W2, b2, g2, be2):
    src = edge_index[0]
    dst = edge_index[1]
    pad = N + (jnp.arange(EPAD - E, dtype=jnp.int32) % (NPAD - N))
    srcp = jnp.concatenate([src, pad])
    dflat = jnp.concatenate([dst, pad])
    dstp = dflat.reshape(ROWS_ALL, CH)

    c32 = _deg_sc(dflat).reshape(32, NPAD)
    h1 = _mm(x, W1, body=_mm0_body)
    dinv, hp1 = _scale(c32, h1)
    S1 = _edge_sc(srcp, dstp, hp1.reshape(2 * NPAD, 128)).reshape(8, NPAD, 128)
    o1, a1, q1 = _epiA(S1, h1, dinv, b1)
    y1 = _epiB(o1, a1, q1, g1, be1)
    h2 = _mm(y1, W2)
    _, hp2 = _scale(c32, h2)
    S2 = _edge_sc(srcp, dstp, hp2.reshape(2 * NPAD, 128)).reshape(8, NPAD, 128)
    o2, a2, q2 = _epiA(S2, h2, dinv, b2)
    return _epiB(o2, a2, q2, g2, be2, out_rows=N)
